# Initial kernel scaffold; baseline (speedup 1.0000x reference)
import jax
import jax.numpy as jnp
from jax.experimental import pallas as pl


def kernel(x, drop_idx):
    raise NotImplementedError("write your pallas kernel here")


# speedup vs baseline: 4.8090x; 4.8090x over previous
import jax
import jax.numpy as jnp
from jax.experimental import pallas as pl
from jax.experimental.pallas import tpu as pltpu

_K = 32
_S = 250
_NDROP = 4


def _body(drop_ref, x_ref, o_ref):
    bd = pl.program_id(1)
    bh = pl.program_id(2)
    v = x_ref[...]
    wpatch = jax.lax.broadcasted_iota(jnp.int32, v.shape, 3) // _K
    mask = None
    for n in range(_NDROP):
        p = drop_ref[n]
        hit = (p // 64 == bd) & ((p // 8) % 8 == bh)
        m = hit & (p % 8 == wpatch)
        mask = m if mask is None else (mask | m)
    o_ref[...] = jnp.where(mask, 0.0, v)


def kernel(x, drop_idx):
    B = x.shape[0]
    xs = x.reshape(B, _S, _S, _S)
    nblk = (_S + _K - 1) // _K
    out = pl.pallas_call(
        _body,
        grid=(B, nblk, nblk),
        in_specs=[
            pl.BlockSpec(memory_space=pltpu.SMEM),
            pl.BlockSpec((1, _K, _K, _S), lambda b, i, j: (b, i, j, 0)),
        ],
        out_specs=pl.BlockSpec((1, _K, _K, _S), lambda b, i, j: (b, i, j, 0)),
        out_shape=jax.ShapeDtypeStruct((B, _S, _S, _S), jnp.float32),
        compiler_params=pltpu.CompilerParams(
            dimension_semantics=("parallel", "parallel", "parallel"),
        ),
    )(drop_idx.astype(jnp.int32), xs)
    return out.reshape(x.shape)
